# baseline (speedup 1.0000x reference)
import dataclasses
import functools

import jax
import jax.numpy as jnp
from jax import lax
from jax.experimental import pallas as pl
from jax.experimental.pallas import tpu as pltpu
from jax.experimental.pallas import tpu_sc as plsc

N_NODES = 10000
E_TOTAL = 320000
D = 128
PE = 327680
PROWS = PE // D
RT = PROWS // 16
N_PH = 5
PR = RT // N_PH
BE = 4096
NB = PE // BE
CH = BE // D
HI = lax.Precision.HIGHEST

_mesh = plsc.VectorSubcoreMesh(core_axis_name="c", subcore_axis_name="s")

_sc_cp = pltpu.CompilerParams()
if "needs_layout_passes" in pltpu.CompilerParams.__dataclass_fields__:
    _sc_cp = dataclasses.replace(_sc_cp, needs_layout_passes=False)


def _leaky(x):
    return jnp.where(x >= 0, x, 0.01 * x)


_CD = (((1,), (0,)), ((), ()))


def _lnT(x, g, b, ones_row):
    n = x.shape[0]
    m = lax.dot_general(ones_row, x, _CD, precision=HI,
                        preferred_element_type=jnp.float32) * (1.0 / n)
    xc = x - m
    v = lax.dot_general(ones_row, xc * xc, _CD, precision=HI,
                        preferred_element_type=jnp.float32) * (1.0 / n)
    return xc * (1.0 / jnp.sqrt(v + 1e-5)) * g + b


def _pre_body(embs_ref, embt_ref, nss_ref, nst_ref):
    es = embs_ref[...]
    et = embt_ref[...]
    nss_ref[...] = jnp.sum(es * es, axis=1)
    nst_ref[...] = jnp.sum(et * et, axis=1)


def _pre(emb_s, emb_t):
    n = emb_s.shape[0]
    return pl.pallas_call(
        _pre_body,
        out_shape=(jax.ShapeDtypeStruct((n,), jnp.float32),
                   jax.ShapeDtypeStruct((n,), jnp.float32)),
    )(emb_s, emb_t)


@functools.partial(
    pl.kernel, mesh=_mesh,
    out_type=(jax.ShapeDtypeStruct((PE, D), jnp.float32),
              jax.ShapeDtypeStruct((PE, D), jnp.float32)),
    scratch_types=[
        pltpu.VMEM_SHARED((N_NODES, D), jnp.float32),
        pltpu.VMEM((PR, D), jnp.int32),
        pltpu.VMEM((D, D), jnp.float32),
        pltpu.VMEM((D, D), jnp.float32),
        pltpu.SemaphoreType.DMA,
        pltpu.SemaphoreType.DMA,
        pltpu.SemaphoreType.DMA,
        pltpu.SemaphoreType.DMA,
        pltpu.SemaphoreType.DMA,
    ],
    compiler_params=_sc_cp,
)
def _gather(tabs_hbm, tabt_hbm, src_hbm, dst_hbm,
            gs_hbm, gt_hbm,
            spm_tab, idx_v, buf_a, buf_b, ga, gb, oa, ob, misc):
    c = lax.axis_index("c")
    s = lax.axis_index("s")

    @pl.when(c == 0)
    def _():
        @pl.when(s == 0)
        def _():
            pltpu.async_copy(tabs_hbm, spm_tab, misc).wait()

    @pl.when(c == 1)
    def _():
        @pl.when(s == 0)
        def _():
            pltpu.async_copy(tabt_hbm, spm_tab, misc).wait()

    plsc.subcore_barrier()

    def run_side(idx_hbm, out_hbm):
        def g_start(j, buf, sem):
            pltpu.make_async_copy(spm_tab.at[idx_v.at[j]], buf, sem).start()

        def g_wait(j, buf, sem):
            pltpu.make_async_copy(spm_tab.at[idx_v.at[j]], buf, sem).wait()

        def o_copy(p, j, buf, sem):
            e0 = (s * RT + p * PR + j) * D
            return pltpu.make_async_copy(buf, out_hbm.at[pl.ds(e0, D)], sem)

        @pl.loop(0, N_PH)
        def _phase(p):
            pltpu.async_copy(idx_hbm.at[s].at[p], idx_v, misc).wait()
            g_start(0, buf_a, ga)
            g_start(1, buf_b, gb)

            @pl.loop(0, PR // 2 - 1)
            def _(i):
                j0 = 2 * i
                j1 = j0 + 1
                g_wait(j0, buf_a, ga)
                o_copy(p, j0, buf_a, oa).start()
                g_wait(j1, buf_b, gb)
                o_copy(p, j1, buf_b, ob).start()
                o_copy(p, j0, buf_a, oa).wait()
                g_start(j0 + 2, buf_a, ga)
                o_copy(p, j1, buf_b, ob).wait()
                g_start(j1 + 2, buf_b, gb)

            j0 = PR - 2
            j1 = PR - 1
            g_wait(j0, buf_a, ga)
            o_copy(p, j0, buf_a, oa).start()
            g_wait(j1, buf_b, gb)
            o_copy(p, j1, buf_b, ob).start()
            o_copy(p, j0, buf_a, oa).wait()
            o_copy(p, j1, buf_b, ob).wait()

    @pl.when(c == 0)
    def _():
        run_side(src_hbm, gs_hbm)

    @pl.when(c == 1)
    def _():
        run_side(dst_hbm, gt_hbm)


@functools.partial(
    pl.kernel, mesh=_mesh,
    out_type=(jax.ShapeDtypeStruct((16, RT, D), jnp.float32),
              jax.ShapeDtypeStruct((16, RT, D), jnp.float32),
              jax.ShapeDtypeStruct((16, RT, D), jnp.float32),
              jax.ShapeDtypeStruct((16, RT, D), jnp.float32)),
    scratch_types=[
        pltpu.VMEM((RT, D), jnp.int32),
        pltpu.VMEM((N_NODES,), jnp.float32),
        pltpu.VMEM((N_NODES,), jnp.float32),
        pltpu.VMEM((RT, D), jnp.float32),
        pltpu.VMEM((RT, D), jnp.float32),
        pltpu.SemaphoreType.DMA,
    ],
    compiler_params=_sc_cp,
)
def _scal(nss_hbm, nst_hbm, x1s_hbm, x1t_hbm, src_hbm, dst_hbm,
          nse_s_hbm, x1e_s_hbm, nse_t_hbm, x1e_t_hbm,
          idx_v, ns_tab, x1_tab, ns_buf, x1_buf, sem):
    c = lax.axis_index("c")
    s = lax.axis_index("s")

    @pl.when(c == 0)
    def _():
        pltpu.async_copy(nss_hbm, ns_tab, sem).wait()
        pltpu.async_copy(x1s_hbm, x1_tab, sem).wait()
        pltpu.async_copy(src_hbm.at[s], idx_v, sem).wait()

    @pl.when(c == 1)
    def _():
        pltpu.async_copy(nst_hbm, ns_tab, sem).wait()
        pltpu.async_copy(x1t_hbm, x1_tab, sem).wait()
        pltpu.async_copy(dst_hbm.at[s], idx_v, sem).wait()

    @pl.loop(0, RT)
    def _row(j):
        for k in range(D // 16):
            sl = pl.ds(k * 16, 16)
            i16 = idx_v[j, sl]
            ns_buf[j, sl] = plsc.load_gather(ns_tab, [i16])
            x1_buf[j, sl] = plsc.load_gather(x1_tab, [i16])

    @pl.when(c == 0)
    def _():
        pltpu.async_copy(ns_buf, nse_s_hbm.at[s], sem).wait()
        pltpu.async_copy(x1_buf, x1e_s_hbm.at[s], sem).wait()

    @pl.when(c == 1)
    def _():
        pltpu.async_copy(ns_buf, nse_t_hbm.at[s], sem).wait()
        pltpu.async_copy(x1_buf, x1e_t_hbm.at[s], sem).wait()


def _bf(x):
    return x.astype(jnp.bfloat16)


def _fnn_body(gs_ref, gt_ref, nss_ref, nst_ref, xs1_ref, xt1_ref,
              ats_ref, att_ref, ew_ref, m8_ref,
              w0a_ref, w0b_ref, w0xc_ref, b0c_ref, g0c_ref, bb0c_ref,
              w1_ref, b1c_ref, g1c_ref, bb1c_ref,
              w2r_ref, b2_ref,
              y_ref):
    i = pl.program_id(0)
    m8 = m8_ref[...]
    asm = jnp.dot(ats_ref[...], m8, precision=HI,
                  preferred_element_type=jnp.float32) * 0.125
    atm = jnp.dot(att_ref[...], m8, precision=HI,
                  preferred_element_type=jnp.float32) * 0.125
    ew = ew_ref[...]
    xs1 = xs1_ref[...]
    xt1 = xt1_ref[...]
    n2 = (nss_ref[...] + nst_ref[...] + xs1 * xs1 + xt1 * xt1
          + asm * asm + atm * atm + ew * ew)
    inv = 1.0 / jnp.maximum(jnp.sqrt(n2), 1e-12)
    es_chunks = []
    et_chunks = []
    s5_chunks = []
    for k in range(CH):
        sl = slice(k * D, (k + 1) * D)
        ik = inv[k:k + 1, :]
        es_chunks.append(lax.transpose(gs_ref[sl, :], (1, 0)) * ik)
        et_chunks.append(lax.transpose(gt_ref[sl, :], (1, 0)) * ik)
        s5_chunks.append(jnp.concatenate(
            [xs1[k:k + 1, :] * ik, xt1[k:k + 1, :] * ik,
             asm[k:k + 1, :] * ik, atm[k:k + 1, :] * ik,
             ew[k:k + 1, :] * ik], axis=0))
    esn = _bf(jnp.concatenate(es_chunks, axis=1))
    etn = _bf(jnp.concatenate(et_chunks, axis=1))
    s5n = _bf(jnp.concatenate(s5_chunks, axis=1))
    ones_row = jnp.ones((1, D), jnp.float32)
    z = (lax.dot_general(w0a_ref[...], esn, _CD,
                         preferred_element_type=jnp.float32)
         + lax.dot_general(w0b_ref[...], etn, _CD,
                           preferred_element_type=jnp.float32)
         + lax.dot_general(w0xc_ref[...], s5n, _CD,
                           preferred_element_type=jnp.float32)
         + b0c_ref[...])
    y1 = _lnT(_leaky(z), g0c_ref[...], bb0c_ref[...], ones_row)
    z2 = lax.dot_general(w1_ref[...], _bf(y1), _CD,
                         preferred_element_type=jnp.float32)
    y2 = _lnT(_leaky(z2 + b1c_ref[...]), g1c_ref[...], bb1c_ref[...],
              ones_row)
    yv = lax.dot_general(w2r_ref[...], _bf(y2), _CD,
                         preferred_element_type=jnp.float32) + b2_ref[...]
    yv = jnp.maximum(yv, 0.0)
    live_rows = E_TOTAL // D
    for k in range(CH):
        sl = slice(k * D, (k + 1) * D)
        live = (CH * i + k) < live_rows
        y_ref[k:k + 1, :] = jnp.where(live, yv[:, sl], 0.0)


def _fnn(gs, gt, nss_e, nst_e, xs1e, xt1e, ats_p, att_p, ew_p, m8,
         w0a, w0b, w0xc, b0c, g0c, bb0c, w1, b1c, g1c, bb1c, w2r, b2):
    rsp = lambda: pl.BlockSpec((CH, D), lambda i: (i, 0))
    wspec = lambda r, c: pl.BlockSpec((r, c), lambda i: (0, 0))
    return pl.pallas_call(
        _fnn_body,
        grid=(NB,),
        in_specs=[
            pl.BlockSpec((BE, D), lambda i: (i, 0)),
            pl.BlockSpec((BE, D), lambda i: (i, 0)),
            rsp(), rsp(), rsp(), rsp(),
            pl.BlockSpec((CH, 1024), lambda i: (i, 0)),
            pl.BlockSpec((CH, 1024), lambda i: (i, 0)),
            rsp(),
            wspec(1024, D),
            wspec(D, D), wspec(D, D),
            wspec(D, 5), wspec(D, 1), wspec(D, 1), wspec(D, 1),
            wspec(D, D), wspec(D, 1), wspec(D, 1), wspec(D, 1),
            wspec(1, D), wspec(1, 1),
        ],
        out_specs=rsp(),
        out_shape=jax.ShapeDtypeStruct((PROWS, D), jnp.float32),
        compiler_params=pltpu.CompilerParams(
            dimension_semantics=("parallel",)),
    )(gs, gt, nss_e, nst_e, xs1e, xt1e, ats_p, att_p, ew_p, m8,
      w0a, w0b, w0xc, b0c, g0c, bb0c, w1, b1c, g1c, bb1c, w2r, b2)


@functools.partial(
    pl.kernel, mesh=_mesh,
    out_type=(jax.ShapeDtypeStruct((16, RT, D), jnp.float32),
              jax.ShapeDtypeStruct((16, RT, D), jnp.float32)),
    scratch_types=[
        pltpu.VMEM((RT, D), jnp.float32),
        pltpu.VMEM((RT, D), jnp.int32),
        pltpu.VMEM_SHARED((N_NODES,), jnp.float32),
        pltpu.VMEM((N_NODES,), jnp.float32),
        pltpu.VMEM((RT, D), jnp.float32),
        pltpu.SemaphoreType.DMA,
    ],
    compiler_params=_sc_cp,
)
def _seg(y_hbm, src_hbm, dst_hbm, gi_hbm, gj_hbm,
         y_v, idx_v, acc_sh, acc_lo, g_v, sem):
    c = lax.axis_index("c")
    s = lax.axis_index("s")

    @pl.when(s == 0)
    def _zero():
        @pl.loop(0, N_NODES // 16)
        def _(i):
            acc_lo[pl.ds(i * 16, 16)] = jnp.zeros((16,), jnp.float32)
        pltpu.async_copy(acc_lo, acc_sh, sem).wait()

    pltpu.async_copy(y_hbm.at[s], y_v, sem).wait()

    @pl.when(c == 0)
    def _():
        pltpu.async_copy(src_hbm.at[s], idx_v, sem).wait()

    @pl.when(c == 1)
    def _():
        pltpu.async_copy(dst_hbm.at[s], idx_v, sem).wait()

    plsc.subcore_barrier()

    @pl.loop(0, RT)
    def _scatter(j):
        pltpu.sync_copy(y_v.at[j], acc_sh.at[idx_v.at[j]], add=True)

    plsc.subcore_barrier()
    pltpu.async_copy(acc_sh, acc_lo, sem).wait()

    @pl.loop(0, RT)
    def _gatherback(j):
        for k in range(D // 16):
            sl = pl.ds(k * 16, 16)
            g_v[j, sl] = plsc.load_gather(acc_lo, [idx_v[j, sl]])

    @pl.when(c == 0)
    def _():
        pltpu.async_copy(g_v, gi_hbm.at[s], sem).wait()

    @pl.when(c == 1)
    def _():
        pltpu.async_copy(g_v, gj_hbm.at[s], sem).wait()


CW = 1024
CROWS = PE // CW
CB = 8


def _coef_body(y_ref, gi_ref, xs_ref, gj_ref, xt_ref,
               fw0_ref, fb0c_ref, fg_ref, fb_ref, fw1r_ref, fb1_ref,
               out_ref):
    fw0 = fw0_ref[...]
    fb0c = fb0c_ref[...]
    fg = fg_ref[...]
    fb = fb_ref[...]
    fw1r = fw1r_ref[...]
    fb1 = fb1_ref[...]
    ones64 = jnp.ones((1, 64), jnp.float32)
    for r in range(CB):
        rs = slice(r, r + 1)
        v = jnp.concatenate([y_ref[rs, :], gi_ref[rs, :], xs_ref[rs, :],
                             gj_ref[rs, :], xt_ref[rs, :]], axis=0)
        h = lax.dot_general(fw0, _bf(v), _CD,
                            preferred_element_type=jnp.float32)
        h = _leaky(h + fb0c)
        hn = _lnT(h, fg, fb, ones64)
        cf = lax.dot_general(fw1r, _bf(hn), _CD,
                             preferred_element_type=jnp.float32) + fb1
        out_ref[rs, :] = y_ref[rs, :] * jnp.maximum(cf, 0.0)


def _coef(y_r, gi_r, xs_r, gj_r, xt_r, fw0, fb0c, fg, fb, fw1r, fb1):
    rsp = lambda: pl.BlockSpec((CB, CW), lambda i: (i, 0))
    wspec = lambda r, c: pl.BlockSpec((r, c), lambda i: (0, 0))
    return pl.pallas_call(
        _coef_body,
        grid=(CROWS // CB,),
        in_specs=[rsp(), rsp(), rsp(), rsp(), rsp(),
                  wspec(64, 5), wspec(64, 1), wspec(64, 1), wspec(64, 1),
                  wspec(1, 64), wspec(1, 1)],
        out_specs=rsp(),
        out_shape=jax.ShapeDtypeStruct((CROWS, CW), jnp.float32),
        compiler_params=pltpu.CompilerParams(
            dimension_semantics=("parallel",)),
    )(y_r, gi_r, xs_r, gj_r, xt_r, fw0, fb0c, fg, fb, fw1r, fb1)


def kernel(emb_s, emb_t, at_s, at_t, x_s, x_t, edge_index, edge_weight,
           W0, b0, W1, b1, W2, b2, ln0_g, ln0_b, ln1_g, ln1_b,
           fW0, fb0, fln_g, fln_b, fW1, fb1):
    pad = PE - E_TOTAL
    zi = jnp.zeros((pad,), jnp.int32)
    zf = jnp.zeros((pad,), jnp.float32)
    srcf = jnp.concatenate([edge_index[0], zi])
    dstf = jnp.concatenate([edge_index[1], zi])
    ns_s, ns_t = _pre(emb_s, emb_t)
    gs, gt = _gather(emb_s, emb_t,
                     srcf.reshape(16, N_PH, PR, D),
                     dstf.reshape(16, N_PH, PR, D))
    src3 = srcf.reshape(16, RT, D)
    dst3 = dstf.reshape(16, RT, D)
    nse_s, x1e_s, nse_t, x1e_t = _scal(ns_s, ns_t, x_s[:, 1], x_t[:, 1],
                                       src3, dst3)
    ats_p = jnp.concatenate([at_s[1].reshape(-1),
                             jnp.zeros((pad * 8,), jnp.float32)])
    att_p = jnp.concatenate([at_t[1].reshape(-1),
                             jnp.zeros((pad * 8,), jnp.float32)])
    ew_p = jnp.concatenate([edge_weight[:, 0], zf]).reshape(PROWS, D)
    m8 = (jnp.arange(1024)[:, None] // 8
          == jnp.arange(D)[None, :]).astype(jnp.float32)
    v2 = lambda a: a.reshape(PROWS, D)
    col = lambda a: a.reshape(D, 1)
    bf16 = jnp.bfloat16
    y = _fnn(gs, gt, v2(nse_s), v2(nse_t), v2(x1e_s), v2(x1e_t),
             ats_p.reshape(PROWS, 1024), att_p.reshape(PROWS, 1024), ew_p, m8,
             W0[:, 0:D].astype(bf16), W0[:, D:2 * D].astype(bf16),
             W0[:, 2 * D:].astype(bf16), col(b0), col(ln0_g), col(ln0_b),
             W1.astype(bf16), col(b1), col(ln1_g), col(ln1_b),
             W2.astype(bf16), b2.reshape(1, 1))
    vc = lambda a: a.reshape(CROWS, CW)
    xs_c = vc(x1e_s)
    xt_c = vc(x1e_t)
    fargs = (fW0.astype(bf16), fb0.reshape(64, 1), fln_g.reshape(64, 1),
             fln_b.reshape(64, 1), fW1.astype(bf16), fb1.reshape(1, 1))
    for _ in range(2):
        gi3, gj3 = _seg(y.reshape(16, RT, D), src3, dst3)
        y = _coef(vc(y), vc(gi3), xs_c, vc(gj3), xt_c, *fargs)
    return y.reshape(PE)[:E_TOTAL]


# speedup vs baseline: 1.1824x; 1.1824x over previous
import dataclasses
import functools

import jax
import jax.numpy as jnp
from jax import lax
from jax.experimental import pallas as pl
from jax.experimental.pallas import tpu as pltpu
from jax.experimental.pallas import tpu_sc as plsc

N_NODES = 10000
E_TOTAL = 320000
D = 128
PE = 327680
PROWS = PE // D
RT = PROWS // 16
HPROWS = PROWS // 2
HRT = HPROWS // 16
N_PH = 4
PR = HRT // N_PH
BE = 4096
NB = PE // BE // 2
CH = BE // D
HI = lax.Precision.HIGHEST

_mesh = plsc.VectorSubcoreMesh(core_axis_name="c", subcore_axis_name="s")

_sc_cp = pltpu.CompilerParams()
if "needs_layout_passes" in pltpu.CompilerParams.__dataclass_fields__:
    _sc_cp = dataclasses.replace(_sc_cp, needs_layout_passes=False)


def _leaky(x):
    return jnp.where(x >= 0, x, 0.01 * x)


_CD = (((1,), (0,)), ((), ()))


def _lnT(x, g, b):
    m = jnp.mean(x, axis=0, keepdims=True)
    xc = x - m
    v = jnp.mean(xc * xc, axis=0, keepdims=True)
    return xc * (1.0 / jnp.sqrt(v + 1e-5)) * g + b


def _pre_body(embs_ref, embt_ref, nss_ref, nst_ref):
    es = embs_ref[...]
    et = embt_ref[...]
    nss_ref[...] = jnp.sum(es * es, axis=1)
    nst_ref[...] = jnp.sum(et * et, axis=1)


def _pre(emb_s, emb_t):
    n = emb_s.shape[0]
    return pl.pallas_call(
        _pre_body,
        out_shape=(jax.ShapeDtypeStruct((n,), jnp.float32),
                   jax.ShapeDtypeStruct((n,), jnp.float32)),
    )(emb_s, emb_t)


@functools.partial(
    pl.kernel, mesh=_mesh,
    out_type=(jax.ShapeDtypeStruct((PE // 2, D), jnp.float32),
              jax.ShapeDtypeStruct((PE // 2, D), jnp.float32)),
    scratch_types=[
        pltpu.VMEM_SHARED((N_NODES, D), jnp.float32),
        pltpu.VMEM((PR, D), jnp.int32),
        pltpu.VMEM((D, D), jnp.float32),
        pltpu.VMEM((D, D), jnp.float32),
        pltpu.SemaphoreType.DMA,
        pltpu.SemaphoreType.DMA,
        pltpu.SemaphoreType.DMA,
        pltpu.SemaphoreType.DMA,
        pltpu.SemaphoreType.DMA,
    ],
    compiler_params=_sc_cp,
)
def _gather(tabs_hbm, tabt_hbm, src_hbm, dst_hbm,
            gs_hbm, gt_hbm,
            spm_tab, idx_v, buf_a, buf_b, ga, gb, oa, ob, misc):
    c = lax.axis_index("c")
    s = lax.axis_index("s")

    @pl.when(c == 0)
    def _():
        @pl.when(s == 0)
        def _():
            pltpu.async_copy(tabs_hbm, spm_tab, misc).wait()

    @pl.when(c == 1)
    def _():
        @pl.when(s == 0)
        def _():
            pltpu.async_copy(tabt_hbm, spm_tab, misc).wait()

    plsc.subcore_barrier()

    def run_side(idx_hbm, out_hbm):
        def g_start(j, buf, sem):
            pltpu.make_async_copy(spm_tab.at[idx_v.at[j]], buf, sem).start()

        def g_wait(j, buf, sem):
            pltpu.make_async_copy(spm_tab.at[idx_v.at[j]], buf, sem).wait()

        def o_copy(p, j, buf, sem):
            e0 = (s * HRT + p * PR + j) * D
            return pltpu.make_async_copy(buf, out_hbm.at[pl.ds(e0, D)], sem)

        @pl.loop(0, N_PH)
        def _phase(p):
            pltpu.async_copy(idx_hbm.at[s].at[p], idx_v, misc).wait()
            g_start(0, buf_a, ga)
            g_start(1, buf_b, gb)

            @pl.loop(0, PR // 2 - 1)
            def _(i):
                j0 = 2 * i
                j1 = j0 + 1
                g_wait(j0, buf_a, ga)
                o_copy(p, j0, buf_a, oa).start()
                g_wait(j1, buf_b, gb)
                o_copy(p, j1, buf_b, ob).start()
                o_copy(p, j0, buf_a, oa).wait()
                g_start(j0 + 2, buf_a, ga)
                o_copy(p, j1, buf_b, ob).wait()
                g_start(j1 + 2, buf_b, gb)

            j0 = PR - 2
            j1 = PR - 1
            g_wait(j0, buf_a, ga)
            o_copy(p, j0, buf_a, oa).start()
            g_wait(j1, buf_b, gb)
            o_copy(p, j1, buf_b, ob).start()
            o_copy(p, j0, buf_a, oa).wait()
            o_copy(p, j1, buf_b, ob).wait()

    @pl.when(c == 0)
    def _():
        run_side(src_hbm, gs_hbm)

    @pl.when(c == 1)
    def _():
        run_side(dst_hbm, gt_hbm)


@functools.partial(
    pl.kernel, mesh=_mesh,
    out_type=(jax.ShapeDtypeStruct((16, RT, D), jnp.float32),
              jax.ShapeDtypeStruct((16, RT, D), jnp.float32),
              jax.ShapeDtypeStruct((16, RT, D), jnp.float32),
              jax.ShapeDtypeStruct((16, RT, D), jnp.float32)),
    scratch_types=[
        pltpu.VMEM((RT, D), jnp.int32),
        pltpu.VMEM((N_NODES,), jnp.float32),
        pltpu.VMEM((N_NODES,), jnp.float32),
        pltpu.VMEM((RT, D), jnp.float32),
        pltpu.VMEM((RT, D), jnp.float32),
        pltpu.SemaphoreType.DMA,
    ],
    compiler_params=_sc_cp,
)
def _scal(nss_hbm, nst_hbm, x1s_hbm, x1t_hbm, src_hbm, dst_hbm,
          nse_s_hbm, x1e_s_hbm, nse_t_hbm, x1e_t_hbm,
          idx_v, ns_tab, x1_tab, ns_buf, x1_buf, sem):
    c = lax.axis_index("c")
    s = lax.axis_index("s")

    @pl.when(c == 0)
    def _():
        pltpu.async_copy(nss_hbm, ns_tab, sem).wait()
        pltpu.async_copy(x1s_hbm, x1_tab, sem).wait()
        pltpu.async_copy(src_hbm.at[s], idx_v, sem).wait()

    @pl.when(c == 1)
    def _():
        pltpu.async_copy(nst_hbm, ns_tab, sem).wait()
        pltpu.async_copy(x1t_hbm, x1_tab, sem).wait()
        pltpu.async_copy(dst_hbm.at[s], idx_v, sem).wait()

    @pl.loop(0, RT)
    def _row(j):
        for k in range(D // 16):
            sl = pl.ds(k * 16, 16)
            i16 = idx_v[j, sl]
            ns_buf[j, sl] = plsc.load_gather(ns_tab, [i16])
            x1_buf[j, sl] = plsc.load_gather(x1_tab, [i16])

    @pl.when(c == 0)
    def _():
        pltpu.async_copy(ns_buf, nse_s_hbm.at[s], sem).wait()
        pltpu.async_copy(x1_buf, x1e_s_hbm.at[s], sem).wait()

    @pl.when(c == 1)
    def _():
        pltpu.async_copy(ns_buf, nse_t_hbm.at[s], sem).wait()
        pltpu.async_copy(x1_buf, x1e_t_hbm.at[s], sem).wait()


def _bf(x):
    return x.astype(jnp.bfloat16)


def _fnn_body(gs_ref, gt_ref, nss_ref, nst_ref, xs1_ref, xt1_ref,
              ats_ref, att_ref, ew_ref, m8_ref,
              w0a_ref, w0b_ref, w0xc_ref, b0c_ref, g0c_ref, bb0c_ref,
              w1_ref, b1c_ref, g1c_ref, bb1c_ref,
              w2r_ref, b2_ref,
              y_ref, *, row0):
    i = pl.program_id(0)
    m8 = m8_ref[...]
    asm = jnp.dot(ats_ref[...], m8, precision=HI,
                  preferred_element_type=jnp.float32) * 0.125
    atm = jnp.dot(att_ref[...], m8, precision=HI,
                  preferred_element_type=jnp.float32) * 0.125
    ew = ew_ref[...]
    xs1 = xs1_ref[...]
    xt1 = xt1_ref[...]
    n2 = (nss_ref[...] + nst_ref[...] + xs1 * xs1 + xt1 * xt1
          + asm * asm + atm * atm + ew * ew)
    inv = 1.0 / jnp.maximum(jnp.sqrt(n2), 1e-12)
    es_chunks = []
    et_chunks = []
    s5_chunks = []
    for k in range(CH):
        sl = slice(k * D, (k + 1) * D)
        ik = inv[k:k + 1, :]
        es_chunks.append(lax.transpose(gs_ref[sl, :], (1, 0)) * ik)
        et_chunks.append(lax.transpose(gt_ref[sl, :], (1, 0)) * ik)
        s5_chunks.append(jnp.concatenate(
            [xs1[k:k + 1, :] * ik, xt1[k:k + 1, :] * ik,
             asm[k:k + 1, :] * ik, atm[k:k + 1, :] * ik,
             ew[k:k + 1, :] * ik], axis=0))
    esn = _bf(jnp.concatenate(es_chunks, axis=1))
    etn = _bf(jnp.concatenate(et_chunks, axis=1))
    s5n = _bf(jnp.concatenate(s5_chunks, axis=1))
    z = (lax.dot_general(w0a_ref[...], esn, _CD,
                         preferred_element_type=jnp.float32)
         + lax.dot_general(w0b_ref[...], etn, _CD,
                           preferred_element_type=jnp.float32)
         + lax.dot_general(w0xc_ref[...], s5n, _CD,
                           preferred_element_type=jnp.float32)
         + b0c_ref[...])
    y1 = _lnT(_leaky(z), g0c_ref[...], bb0c_ref[...])
    z2 = lax.dot_general(w1_ref[...], _bf(y1), _CD,
                         preferred_element_type=jnp.float32)
    y2 = _lnT(_leaky(z2 + b1c_ref[...]), g1c_ref[...], bb1c_ref[...])
    yv = lax.dot_general(w2r_ref[...], _bf(y2), _CD,
                         preferred_element_type=jnp.float32) + b2_ref[...]
    yv = jnp.maximum(yv, 0.0)
    live_rows = E_TOTAL // D
    for k in range(CH):
        sl = slice(k * D, (k + 1) * D)
        live = (row0 + CH * i + k) < live_rows
        y_ref[k:k + 1, :] = jnp.where(live, yv[:, sl], 0.0)


def _fnn(row0, gs, gt, nss_e, nst_e, xs1e, xt1e, ats_p, att_p, ew_p, m8,
         w0a, w0b, w0xc, b0c, g0c, bb0c, w1, b1c, g1c, bb1c, w2r, b2):
    rsp = lambda: pl.BlockSpec((CH, D), lambda i: (i, 0))
    wspec = lambda r, c: pl.BlockSpec((r, c), lambda i: (0, 0))
    return pl.pallas_call(
        functools.partial(_fnn_body, row0=row0),
        grid=(NB,),
        in_specs=[
            pl.BlockSpec((BE, D), lambda i: (i, 0)),
            pl.BlockSpec((BE, D), lambda i: (i, 0)),
            rsp(), rsp(), rsp(), rsp(),
            pl.BlockSpec((CH, 1024), lambda i: (i, 0)),
            pl.BlockSpec((CH, 1024), lambda i: (i, 0)),
            rsp(),
            wspec(1024, D),
            wspec(D, D), wspec(D, D),
            wspec(D, 5), wspec(D, 1), wspec(D, 1), wspec(D, 1),
            wspec(D, D), wspec(D, 1), wspec(D, 1), wspec(D, 1),
            wspec(1, D), wspec(1, 1),
        ],
        out_specs=rsp(),
        out_shape=jax.ShapeDtypeStruct((HPROWS, D), jnp.float32),
        compiler_params=pltpu.CompilerParams(
            dimension_semantics=("parallel",)),
    )(gs, gt, nss_e, nst_e, xs1e, xt1e, ats_p, att_p, ew_p, m8,
      w0a, w0b, w0xc, b0c, g0c, bb0c, w1, b1c, g1c, bb1c, w2r, b2)


@functools.partial(
    pl.kernel, mesh=_mesh,
    out_type=(jax.ShapeDtypeStruct((16, RT, D), jnp.float32),
              jax.ShapeDtypeStruct((16, RT, D), jnp.float32)),
    scratch_types=[
        pltpu.VMEM((RT, D), jnp.float32),
        pltpu.VMEM((RT, D), jnp.int32),
        pltpu.VMEM_SHARED((N_NODES,), jnp.float32),
        pltpu.VMEM((N_NODES,), jnp.float32),
        pltpu.VMEM((RT, D), jnp.float32),
        pltpu.SemaphoreType.DMA,
    ],
    compiler_params=_sc_cp,
)
def _seg(y_hbm, src_hbm, dst_hbm, gi_hbm, gj_hbm,
         y_v, idx_v, acc_sh, acc_lo, g_v, sem):
    c = lax.axis_index("c")
    s = lax.axis_index("s")

    @pl.when(s == 0)
    def _zero():
        @pl.loop(0, N_NODES // 16)
        def _(i):
            acc_lo[pl.ds(i * 16, 16)] = jnp.zeros((16,), jnp.float32)
        pltpu.async_copy(acc_lo, acc_sh, sem).wait()

    pltpu.async_copy(y_hbm.at[s], y_v, sem).wait()

    @pl.when(c == 0)
    def _():
        pltpu.async_copy(src_hbm.at[s], idx_v, sem).wait()

    @pl.when(c == 1)
    def _():
        pltpu.async_copy(dst_hbm.at[s], idx_v, sem).wait()

    plsc.subcore_barrier()

    @pl.loop(0, RT)
    def _scatter(j):
        pltpu.sync_copy(y_v.at[j], acc_sh.at[idx_v.at[j]], add=True)

    plsc.subcore_barrier()
    pltpu.async_copy(acc_sh, acc_lo, sem).wait()

    @pl.loop(0, RT)
    def _gatherback(j):
        for k in range(D // 16):
            sl = pl.ds(k * 16, 16)
            g_v[j, sl] = plsc.load_gather(acc_lo, [idx_v[j, sl]])

    @pl.when(c == 0)
    def _():
        pltpu.async_copy(g_v, gi_hbm.at[s], sem).wait()

    @pl.when(c == 1)
    def _():
        pltpu.async_copy(g_v, gj_hbm.at[s], sem).wait()


CW = 1024
CROWS = PE // CW
CB = 8


def _coef_body(y_ref, gi_ref, xs_ref, gj_ref, xt_ref,
               fw0_ref, fb0c_ref, fg_ref, fb_ref, fw1r_ref, fb1_ref,
               out_ref):
    fw0 = fw0_ref[...]
    fb0c = fb0c_ref[...]
    fg = fg_ref[...]
    fb = fb_ref[...]
    fw1r = fw1r_ref[...]
    fb1 = fb1_ref[...]
    for r in range(CB):
        rs = slice(r, r + 1)
        v = jnp.concatenate([y_ref[rs, :], gi_ref[rs, :], xs_ref[rs, :],
                             gj_ref[rs, :], xt_ref[rs, :]], axis=0)
        h = lax.dot_general(fw0, _bf(v), _CD,
                            preferred_element_type=jnp.float32)
        h = _leaky(h + fb0c)
        hn = _lnT(h, fg, fb)
        cf = lax.dot_general(fw1r, _bf(hn), _CD,
                             preferred_element_type=jnp.float32) + fb1
        out_ref[rs, :] = y_ref[rs, :] * jnp.maximum(cf, 0.0)


def _coef(y_r, gi_r, xs_r, gj_r, xt_r, fw0, fb0c, fg, fb, fw1r, fb1):
    rsp = lambda: pl.BlockSpec((CB, CW), lambda i: (i, 0))
    wspec = lambda r, c: pl.BlockSpec((r, c), lambda i: (0, 0))
    return pl.pallas_call(
        _coef_body,
        grid=(CROWS // CB,),
        in_specs=[rsp(), rsp(), rsp(), rsp(), rsp(),
                  wspec(64, 5), wspec(64, 1), wspec(64, 1), wspec(64, 1),
                  wspec(1, 64), wspec(1, 1)],
        out_specs=rsp(),
        out_shape=jax.ShapeDtypeStruct((CROWS, CW), jnp.float32),
        compiler_params=pltpu.CompilerParams(
            dimension_semantics=("parallel",)),
    )(y_r, gi_r, xs_r, gj_r, xt_r, fw0, fb0c, fg, fb, fw1r, fb1)


def kernel(emb_s, emb_t, at_s, at_t, x_s, x_t, edge_index, edge_weight,
           W0, b0, W1, b1, W2, b2, ln0_g, ln0_b, ln1_g, ln1_b,
           fW0, fb0, fln_g, fln_b, fW1, fb1):
    pad = PE - E_TOTAL
    zi = jnp.zeros((pad,), jnp.int32)
    zf = jnp.zeros((pad,), jnp.float32)
    srcf = jnp.concatenate([edge_index[0], zi])
    dstf = jnp.concatenate([edge_index[1], zi])
    ns_s, ns_t = _pre(emb_s, emb_t)
    src3 = srcf.reshape(16, RT, D)
    dst3 = dstf.reshape(16, RT, D)
    nse_s, x1e_s, nse_t, x1e_t = _scal(ns_s, ns_t, x_s[:, 1], x_t[:, 1],
                                       src3, dst3)
    srch = srcf.reshape(2, 16, N_PH, PR, D)
    dsth = dstf.reshape(2, 16, N_PH, PR, D)
    ats_p = jnp.concatenate([at_s[1].reshape(-1),
                             jnp.zeros((pad * 8,), jnp.float32)])
    att_p = jnp.concatenate([at_t[1].reshape(-1),
                             jnp.zeros((pad * 8,), jnp.float32)])
    ats_h = ats_p.reshape(2, HPROWS, 1024)
    att_h = att_p.reshape(2, HPROWS, 1024)
    ew_h = jnp.concatenate([edge_weight[:, 0], zf]).reshape(2, HPROWS, D)
    m8 = (jnp.arange(1024)[:, None] // 8
          == jnp.arange(D)[None, :]).astype(jnp.float32)
    vh = lambda a: a.reshape(2, HPROWS, D)
    col = lambda a: a.reshape(D, 1)
    bf16 = jnp.bfloat16
    nse_sh, nse_th = vh(nse_s), vh(nse_t)
    x1e_sh, x1e_th = vh(x1e_s), vh(x1e_t)
    wargs = (W0[:, 0:D].astype(bf16), W0[:, D:2 * D].astype(bf16),
             W0[:, 2 * D:].astype(bf16), col(b0), col(ln0_g), col(ln0_b),
             W1.astype(bf16), col(b1), col(ln1_g), col(ln1_b),
             W2.astype(bf16), b2.reshape(1, 1))
    y_halves = []
    for h in range(2):
        gs_h, gt_h = _gather(emb_s, emb_t, srch[h], dsth[h])
        y_halves.append(_fnn(
            h * HPROWS, gs_h, gt_h, nse_sh[h], nse_th[h],
            x1e_sh[h], x1e_th[h], ats_h[h], att_h[h], ew_h[h], m8, *wargs))
    y = jnp.concatenate(y_halves, axis=0)
    vc = lambda a: a.reshape(CROWS, CW)
    xs_c = vc(x1e_s)
    xt_c = vc(x1e_t)
    fargs = (fW0.astype(bf16), fb0.reshape(64, 1), fln_g.reshape(64, 1),
             fln_b.reshape(64, 1), fW1.astype(bf16), fb1.reshape(1, 1))
    for _ in range(2):
        gi3, gj3 = _seg(y.reshape(16, RT, D), src3, dst3)
        y = _coef(vc(y), vc(gi3), xs_c, vc(gj3), xt_c, *fargs)
    return y.reshape(PE)[:E_TOTAL]


# speedup vs baseline: 1.4475x; 1.2242x over previous
import dataclasses
import functools

import jax
import jax.numpy as jnp
from jax import lax
from jax.experimental import pallas as pl
from jax.experimental.pallas import tpu as pltpu
from jax.experimental.pallas import tpu_sc as plsc

N_NODES = 10000
E_TOTAL = 320000
D = 128
PE = 327680
PROWS = PE // D
RT = PROWS // 16
N_PH = 5
PR = RT // N_PH
BE = 4096
NB = PE // BE
CH = BE // D
HI = lax.Precision.HIGHEST

_mesh = plsc.VectorSubcoreMesh(core_axis_name="c", subcore_axis_name="s")

_sc_cp = pltpu.CompilerParams()
if "needs_layout_passes" in pltpu.CompilerParams.__dataclass_fields__:
    _sc_cp = dataclasses.replace(_sc_cp, needs_layout_passes=False)


def _leaky(x):
    return jnp.where(x >= 0, x, 0.01 * x)


_CD = (((1,), (0,)), ((), ()))


def _lnT(x, g, b):
    m = jnp.mean(x, axis=0, keepdims=True)
    xc = x - m
    v = jnp.mean(xc * xc, axis=0, keepdims=True)
    return xc * (1.0 / jnp.sqrt(v + 1e-5)) * g + b


def _pre_body(embs_ref, embt_ref, nss_ref, nst_ref):
    es = embs_ref[...]
    et = embt_ref[...]
    nss_ref[...] = jnp.sum(es * es, axis=1)
    nst_ref[...] = jnp.sum(et * et, axis=1)


def _pre(emb_s, emb_t):
    n = emb_s.shape[0]
    return pl.pallas_call(
        _pre_body,
        out_shape=(jax.ShapeDtypeStruct((n,), jnp.float32),
                   jax.ShapeDtypeStruct((n,), jnp.float32)),
    )(emb_s, emb_t)


@functools.partial(
    pl.kernel, mesh=_mesh,
    out_type=(jax.ShapeDtypeStruct((PE, D), jnp.float32),
              jax.ShapeDtypeStruct((PE, D), jnp.float32)),
    scratch_types=[
        pltpu.VMEM_SHARED((N_NODES, D), jnp.float32),
        pltpu.VMEM((PR, D), jnp.int32),
        pltpu.VMEM((D, D), jnp.float32),
        pltpu.VMEM((D, D), jnp.float32),
        pltpu.SemaphoreType.DMA,
        pltpu.SemaphoreType.DMA,
        pltpu.SemaphoreType.DMA,
        pltpu.SemaphoreType.DMA,
        pltpu.SemaphoreType.DMA,
    ],
    compiler_params=_sc_cp,
)
def _gather(tabs_hbm, tabt_hbm, src_hbm, dst_hbm,
            gs_hbm, gt_hbm,
            spm_tab, idx_v, buf_a, buf_b, ga, gb, oa, ob, misc):
    c = lax.axis_index("c")
    s = lax.axis_index("s")

    @pl.when(c == 0)
    def _():
        @pl.when(s == 0)
        def _():
            pltpu.async_copy(tabs_hbm, spm_tab, misc).wait()

    @pl.when(c == 1)
    def _():
        @pl.when(s == 0)
        def _():
            pltpu.async_copy(tabt_hbm, spm_tab, misc).wait()

    plsc.subcore_barrier()

    def run_side(idx_hbm, out_hbm):
        def g_start(j, buf, sem):
            pltpu.make_async_copy(spm_tab.at[idx_v.at[j]], buf, sem).start()

        def g_wait(j, buf, sem):
            pltpu.make_async_copy(spm_tab.at[idx_v.at[j]], buf, sem).wait()

        def o_copy(p, j, buf, sem):
            e0 = (s * RT + p * PR + j) * D
            return pltpu.make_async_copy(buf, out_hbm.at[pl.ds(e0, D)], sem)

        @pl.loop(0, N_PH)
        def _phase(p):
            pltpu.async_copy(idx_hbm.at[s].at[p], idx_v, misc).wait()
            g_start(0, buf_a, ga)
            g_start(1, buf_b, gb)

            @pl.loop(0, PR // 2 - 1)
            def _(i):
                j0 = 2 * i
                j1 = j0 + 1
                g_wait(j0, buf_a, ga)
                o_copy(p, j0, buf_a, oa).start()
                g_wait(j1, buf_b, gb)
                o_copy(p, j1, buf_b, ob).start()
                o_copy(p, j0, buf_a, oa).wait()
                g_start(j0 + 2, buf_a, ga)
                o_copy(p, j1, buf_b, ob).wait()
                g_start(j1 + 2, buf_b, gb)

            j0 = PR - 2
            j1 = PR - 1
            g_wait(j0, buf_a, ga)
            o_copy(p, j0, buf_a, oa).start()
            g_wait(j1, buf_b, gb)
            o_copy(p, j1, buf_b, ob).start()
            o_copy(p, j0, buf_a, oa).wait()
            o_copy(p, j1, buf_b, ob).wait()

    @pl.when(c == 0)
    def _():
        run_side(src_hbm, gs_hbm)

    @pl.when(c == 1)
    def _():
        run_side(dst_hbm, gt_hbm)


@functools.partial(
    pl.kernel, mesh=_mesh,
    out_type=(jax.ShapeDtypeStruct((16, RT, D), jnp.float32),
              jax.ShapeDtypeStruct((16, RT, D), jnp.float32),
              jax.ShapeDtypeStruct((16, RT, D), jnp.float32),
              jax.ShapeDtypeStruct((16, RT, D), jnp.float32)),
    scratch_types=[
        pltpu.VMEM((RT, D), jnp.int32),
        pltpu.VMEM((N_NODES,), jnp.float32),
        pltpu.VMEM((N_NODES,), jnp.float32),
        pltpu.VMEM((RT, D), jnp.float32),
        pltpu.VMEM((RT, D), jnp.float32),
        pltpu.SemaphoreType.DMA,
    ],
    compiler_params=_sc_cp,
)
def _scal(nss_hbm, nst_hbm, x1s_hbm, x1t_hbm, src_hbm, dst_hbm,
          nse_s_hbm, x1e_s_hbm, nse_t_hbm, x1e_t_hbm,
          idx_v, ns_tab, x1_tab, ns_buf, x1_buf, sem):
    c = lax.axis_index("c")
    s = lax.axis_index("s")

    @pl.when(c == 0)
    def _():
        pltpu.async_copy(nss_hbm, ns_tab, sem).wait()
        pltpu.async_copy(x1s_hbm, x1_tab, sem).wait()
        pltpu.async_copy(src_hbm.at[s], idx_v, sem).wait()

    @pl.when(c == 1)
    def _():
        pltpu.async_copy(nst_hbm, ns_tab, sem).wait()
        pltpu.async_copy(x1t_hbm, x1_tab, sem).wait()
        pltpu.async_copy(dst_hbm.at[s], idx_v, sem).wait()

    @pl.loop(0, RT)
    def _row(j):
        for k in range(D // 16):
            sl = pl.ds(k * 16, 16)
            i16 = idx_v[j, sl]
            ns_buf[j, sl] = plsc.load_gather(ns_tab, [i16])
            x1_buf[j, sl] = plsc.load_gather(x1_tab, [i16])

    @pl.when(c == 0)
    def _():
        pltpu.async_copy(ns_buf, nse_s_hbm.at[s], sem).wait()
        pltpu.async_copy(x1_buf, x1e_s_hbm.at[s], sem).wait()

    @pl.when(c == 1)
    def _():
        pltpu.async_copy(ns_buf, nse_t_hbm.at[s], sem).wait()
        pltpu.async_copy(x1_buf, x1e_t_hbm.at[s], sem).wait()


def _bf(x):
    return x.astype(jnp.bfloat16)


def _fnn_body(gs_ref, gt_ref, nss_ref, nst_ref, xs1_ref, xt1_ref,
              ats_ref, att_ref, ew_ref, m8_ref,
              w0a_ref, w0b_ref, w0xc_ref, b0c_ref, g0c_ref, bb0c_ref,
              w1_ref, b1c_ref, g1c_ref, bb1c_ref,
              w2r_ref, b2_ref,
              y_ref, *, row0):
    i = pl.program_id(0)
    m8 = m8_ref[...]
    asm = jnp.dot(ats_ref[...], m8, precision=HI,
                  preferred_element_type=jnp.float32) * 0.125
    atm = jnp.dot(att_ref[...], m8, precision=HI,
                  preferred_element_type=jnp.float32) * 0.125
    ew = ew_ref[...]
    xs1 = xs1_ref[...]
    xt1 = xt1_ref[...]
    n2 = (nss_ref[...] + nst_ref[...] + xs1 * xs1 + xt1 * xt1
          + asm * asm + atm * atm + ew * ew)
    inv = 1.0 / jnp.maximum(jnp.sqrt(n2), 1e-12)
    w0xc = w0xc_ref[...]
    es_chunks = []
    et_chunks = []
    zx_chunks = []
    for k in range(CH):
        sl = slice(k * D, (k + 1) * D)
        ik = inv[k:k + 1, :]
        es_chunks.append(lax.transpose(gs_ref[sl, :], (1, 0)) * ik)
        et_chunks.append(lax.transpose(gt_ref[sl, :], (1, 0)) * ik)
        zx = jnp.zeros((D, D), jnp.float32)
        for j, sc in enumerate((xs1, xt1, asm, atm, ew)):
            scn = _bf(sc[k:k + 1, :] * ik).astype(jnp.float32)
            zx = zx + w0xc[:, j:j + 1].astype(jnp.float32) * scn
        zx_chunks.append(zx)
    esn = _bf(jnp.concatenate(es_chunks, axis=1))
    etn = _bf(jnp.concatenate(et_chunks, axis=1))
    zx = jnp.concatenate(zx_chunks, axis=1)
    z = (lax.dot_general(w0a_ref[...], esn, _CD,
                         preferred_element_type=jnp.float32)
         + lax.dot_general(w0b_ref[...], etn, _CD,
                           preferred_element_type=jnp.float32)
         + zx + b0c_ref[...])
    y1 = _lnT(_leaky(z), g0c_ref[...], bb0c_ref[...])
    z2 = lax.dot_general(w1_ref[...], _bf(y1), _CD,
                         preferred_element_type=jnp.float32)
    y2 = _lnT(_leaky(z2 + b1c_ref[...]), g1c_ref[...], bb1c_ref[...])
    yv = lax.dot_general(w2r_ref[...], _bf(y2), _CD,
                         preferred_element_type=jnp.float32) + b2_ref[...]
    yv = jnp.maximum(yv, 0.0)
    live_rows = E_TOTAL // D
    for k in range(CH):
        sl = slice(k * D, (k + 1) * D)
        live = (row0 + CH * i + k) < live_rows
        y_ref[k:k + 1, :] = jnp.where(live, yv[:, sl], 0.0)


def _fnn(row0, gs, gt, nss_e, nst_e, xs1e, xt1e, ats_p, att_p, ew_p, m8,
         w0a, w0b, w0xc, b0c, g0c, bb0c, w1, b1c, g1c, bb1c, w2r, b2):
    rsp = lambda: pl.BlockSpec((CH, D), lambda i: (i, 0))
    wspec = lambda r, c: pl.BlockSpec((r, c), lambda i: (0, 0))
    return pl.pallas_call(
        functools.partial(_fnn_body, row0=row0),
        grid=(NB,),
        in_specs=[
            pl.BlockSpec((BE, D), lambda i: (i, 0)),
            pl.BlockSpec((BE, D), lambda i: (i, 0)),
            rsp(), rsp(), rsp(), rsp(),
            pl.BlockSpec((CH, 1024), lambda i: (i, 0)),
            pl.BlockSpec((CH, 1024), lambda i: (i, 0)),
            rsp(),
            wspec(1024, D),
            wspec(D, D), wspec(D, D),
            wspec(D, 5), wspec(D, 1), wspec(D, 1), wspec(D, 1),
            wspec(D, D), wspec(D, 1), wspec(D, 1), wspec(D, 1),
            wspec(1, D), wspec(1, 1),
        ],
        out_specs=rsp(),
        out_shape=jax.ShapeDtypeStruct((PROWS, D), jnp.float32),
        compiler_params=pltpu.CompilerParams(
            dimension_semantics=("parallel",)),
    )(gs, gt, nss_e, nst_e, xs1e, xt1e, ats_p, att_p, ew_p, m8,
      w0a, w0b, w0xc, b0c, g0c, bb0c, w1, b1c, g1c, bb1c, w2r, b2)


@functools.partial(
    pl.kernel, mesh=_mesh,
    out_type=(jax.ShapeDtypeStruct((16, RT, D), jnp.float32),
              jax.ShapeDtypeStruct((16, RT, D), jnp.float32)),
    scratch_types=[
        pltpu.VMEM((RT, D), jnp.float32),
        pltpu.VMEM((RT, D), jnp.int32),
        pltpu.VMEM_SHARED((N_NODES,), jnp.float32),
        pltpu.VMEM((N_NODES,), jnp.float32),
        pltpu.VMEM((RT, D), jnp.float32),
        pltpu.SemaphoreType.DMA,
    ],
    compiler_params=_sc_cp,
)
def _seg(y_hbm, src_hbm, dst_hbm, gi_hbm, gj_hbm,
         y_v, idx_v, acc_sh, acc_lo, g_v, sem):
    c = lax.axis_index("c")
    s = lax.axis_index("s")

    @pl.when(s == 0)
    def _zero():
        @pl.loop(0, N_NODES // 16)
        def _(i):
            acc_lo[pl.ds(i * 16, 16)] = jnp.zeros((16,), jnp.float32)
        pltpu.async_copy(acc_lo, acc_sh, sem).wait()

    pltpu.async_copy(y_hbm.at[s], y_v, sem).wait()

    @pl.when(c == 0)
    def _():
        pltpu.async_copy(src_hbm.at[s], idx_v, sem).wait()

    @pl.when(c == 1)
    def _():
        pltpu.async_copy(dst_hbm.at[s], idx_v, sem).wait()

    plsc.subcore_barrier()

    @pl.loop(0, RT)
    def _scatter(j):
        pltpu.sync_copy(y_v.at[j], acc_sh.at[idx_v.at[j]], add=True)

    plsc.subcore_barrier()
    pltpu.async_copy(acc_sh, acc_lo, sem).wait()

    @pl.loop(0, RT)
    def _gatherback(j):
        for k in range(D // 16):
            sl = pl.ds(k * 16, 16)
            g_v[j, sl] = plsc.load_gather(acc_lo, [idx_v[j, sl]])

    @pl.when(c == 0)
    def _():
        pltpu.async_copy(g_v, gi_hbm.at[s], sem).wait()

    @pl.when(c == 1)
    def _():
        pltpu.async_copy(g_v, gj_hbm.at[s], sem).wait()


CW = 1024
CROWS = PE // CW
CB = 8


def _coef_body(y_ref, gi_ref, xs_ref, gj_ref, xt_ref,
               fw0_ref, fb0c_ref, fg_ref, fb_ref, fw1r_ref, fb1_ref,
               out_ref):
    fw0 = fw0_ref[...]
    fb0c = fb0c_ref[...]
    fg = fg_ref[...]
    fb = fb_ref[...]
    fw1r = fw1r_ref[...]
    fb1 = fb1_ref[...]
    for r in range(CB):
        rs = slice(r, r + 1)
        v = jnp.concatenate([y_ref[rs, :], gi_ref[rs, :], xs_ref[rs, :],
                             gj_ref[rs, :], xt_ref[rs, :]], axis=0)
        h = lax.dot_general(fw0, _bf(v), _CD,
                            preferred_element_type=jnp.float32)
        h = _leaky(h + fb0c)
        hn = _lnT(h, fg, fb)
        cf = lax.dot_general(fw1r, _bf(hn), _CD,
                             preferred_element_type=jnp.float32) + fb1
        out_ref[rs, :] = y_ref[rs, :] * jnp.maximum(cf, 0.0)


def _coef(y_r, gi_r, xs_r, gj_r, xt_r, fw0, fb0c, fg, fb, fw1r, fb1):
    rsp = lambda: pl.BlockSpec((CB, CW), lambda i: (i, 0))
    wspec = lambda r, c: pl.BlockSpec((r, c), lambda i: (0, 0))
    return pl.pallas_call(
        _coef_body,
        grid=(CROWS // CB,),
        in_specs=[rsp(), rsp(), rsp(), rsp(), rsp(),
                  wspec(64, 5), wspec(64, 1), wspec(64, 1), wspec(64, 1),
                  wspec(1, 64), wspec(1, 1)],
        out_specs=rsp(),
        out_shape=jax.ShapeDtypeStruct((CROWS, CW), jnp.float32),
        compiler_params=pltpu.CompilerParams(
            dimension_semantics=("parallel",)),
    )(y_r, gi_r, xs_r, gj_r, xt_r, fw0, fb0c, fg, fb, fw1r, fb1)


def kernel(emb_s, emb_t, at_s, at_t, x_s, x_t, edge_index, edge_weight,
           W0, b0, W1, b1, W2, b2, ln0_g, ln0_b, ln1_g, ln1_b,
           fW0, fb0, fln_g, fln_b, fW1, fb1):
    pad = PE - E_TOTAL
    zi = jnp.zeros((pad,), jnp.int32)
    zf = jnp.zeros((pad,), jnp.float32)
    srcf = jnp.concatenate([edge_index[0], zi])
    dstf = jnp.concatenate([edge_index[1], zi])
    ns_s, ns_t = _pre(emb_s, emb_t)
    src3 = srcf.reshape(16, RT, D)
    dst3 = dstf.reshape(16, RT, D)
    nse_s, x1e_s, nse_t, x1e_t = _scal(ns_s, ns_t, x_s[:, 1], x_t[:, 1],
                                       src3, dst3)
    gs, gt = _gather(emb_s, emb_t,
                     srcf.reshape(16, N_PH, PR, D),
                     dstf.reshape(16, N_PH, PR, D))
    ats_p = jnp.concatenate([at_s[1].reshape(-1),
                             jnp.zeros((pad * 8,), jnp.float32)])
    att_p = jnp.concatenate([at_t[1].reshape(-1),
                             jnp.zeros((pad * 8,), jnp.float32)])
    ew_p = jnp.concatenate([edge_weight[:, 0], zf]).reshape(PROWS, D)
    m8 = (jnp.arange(1024)[:, None] // 8
          == jnp.arange(D)[None, :]).astype(jnp.float32)
    v2 = lambda a: a.reshape(PROWS, D)
    col = lambda a: a.reshape(D, 1)
    bf16 = jnp.bfloat16
    y = _fnn(0, gs, gt, v2(nse_s), v2(nse_t), v2(x1e_s), v2(x1e_t),
             ats_p.reshape(PROWS, 1024), att_p.reshape(PROWS, 1024), ew_p, m8,
             W0[:, 0:D].astype(bf16), W0[:, D:2 * D].astype(bf16),
             W0[:, 2 * D:].astype(bf16), col(b0), col(ln0_g), col(ln0_b),
             W1.astype(bf16), col(b1), col(ln1_g), col(ln1_b),
             W2.astype(bf16), b2.reshape(1, 1))
    vc = lambda a: a.reshape(CROWS, CW)
    xs_c = vc(x1e_s)
    xt_c = vc(x1e_t)
    fargs = (fW0.astype(bf16), fb0.reshape(64, 1), fln_g.reshape(64, 1),
             fln_b.reshape(64, 1), fW1.astype(bf16), fb1.reshape(1, 1))
    for _ in range(2):
        gi3, gj3 = _seg(y.reshape(16, RT, D), src3, dst3)
        y = _coef(vc(y), vc(gi3), xs_c, vc(gj3), xt_c, *fargs)
    return y.reshape(PE)[:E_TOTAL]
